# baseline (speedup 1.0000x reference)
import functools

import jax
import jax.numpy as jnp
from jax import lax
from jax.experimental import pallas as pl
from jax.experimental.pallas import tpu as pltpu
from jax.experimental.pallas import tpu_sc as plsc

B = 16384
EMB = 64
IN_DIM = 3 * EMB
H1 = 128
H2 = 64

_NC = 2
_NS = 16
_NW = _NC * _NS
_BPW = B // _NW
_CHUNK = 128
_NCHUNK = _BPW // _CHUNK


def _sc_gather_body(uidx, iidx, gidx, utab, itab, gtab, out_u, out_i, out_g,
                    uix_v, iix_v, gix_v, rows_a, rows_b,
                    gsem_a, gsem_b, wsem_a, wsem_b):
    wid = lax.axis_index("s") * _NC + lax.axis_index("c")
    base = wid * _BPW
    pltpu.sync_copy(uidx.at[pl.ds(base, _BPW)], uix_v)
    pltpu.sync_copy(iidx.at[pl.ds(base, _BPW)], iix_v)
    pltpu.sync_copy(gidx.at[pl.ds(base, _BPW)], gix_v)

    work = []
    for tab, ixv, out in ((utab, uix_v, out_u), (itab, iix_v, out_i),
                          (gtab, gix_v, out_g)):
        for j in range(_NCHUNK):
            work.append((tab, ixv, out, j))

    bufs = (rows_a, rows_b)
    gsems = (gsem_a, gsem_b)
    wsems = (wsem_a, wsem_b)
    pending = [None, None]
    for k, (tab, ixv, out, j) in enumerate(work):
        slot = k % 2
        if pending[slot] is not None:
            pending[slot].wait()
            pending[slot] = None
        idx_slice = ixv.at[pl.ds(j * _CHUNK, _CHUNK)]
        pltpu.async_copy(tab.at[idx_slice], bufs[slot], gsems[slot]).wait()
        pending[slot] = pltpu.async_copy(
            bufs[slot], out.at[pl.ds(base + j * _CHUNK, _CHUNK)], wsems[slot])
    for p in pending:
        if p is not None:
            p.wait()


_SC_GATHER = functools.partial(
    pl.kernel,
    out_type=[jax.ShapeDtypeStruct((B, EMB), jnp.float32)] * 3,
    mesh=plsc.VectorSubcoreMesh(core_axis_name="c", subcore_axis_name="s"),
    compiler_params=pltpu.CompilerParams(use_tc_tiling_on_sc=False),
    scratch_types=[
        pltpu.VMEM((_BPW,), jnp.int32),
        pltpu.VMEM((_BPW,), jnp.int32),
        pltpu.VMEM((_BPW,), jnp.int32),
        pltpu.VMEM((_CHUNK, EMB), jnp.float32),
        pltpu.VMEM((_CHUNK, EMB), jnp.float32),
        pltpu.SemaphoreType.DMA,
        pltpu.SemaphoreType.DMA,
        pltpu.SemaphoreType.DMA,
        pltpu.SemaphoreType.DMA,
    ],
)(_sc_gather_body)


_BLK = 2048


def _mlp_body(u_ref, i_ref, g_ref, w1_ref, b1_ref, w2_ref, b2_ref, o_ref):
    x = jnp.concatenate([u_ref[...], i_ref[...], g_ref[...]], axis=1)
    h = jnp.dot(x, w1_ref[...], preferred_element_type=jnp.float32,
                precision=jax.lax.Precision.HIGHEST) + b1_ref[...]
    h = jnp.maximum(h, 0.0)
    o = jnp.dot(h, w2_ref[...], preferred_element_type=jnp.float32,
                precision=jax.lax.Precision.HIGHEST) + b2_ref[...]
    o_ref[...] = jnp.maximum(o, 0.0)


def _tc_mlp(u, i, g, W1, b1, W2, b2):
    return pl.pallas_call(
        _mlp_body,
        grid=(B // _BLK,),
        in_specs=[
            pl.BlockSpec((_BLK, EMB), lambda n: (n, 0)),
            pl.BlockSpec((_BLK, EMB), lambda n: (n, 0)),
            pl.BlockSpec((_BLK, EMB), lambda n: (n, 0)),
            pl.BlockSpec((IN_DIM, H1), lambda n: (0, 0)),
            pl.BlockSpec((1, H1), lambda n: (0, 0)),
            pl.BlockSpec((H1, H2), lambda n: (0, 0)),
            pl.BlockSpec((1, H2), lambda n: (0, 0)),
        ],
        out_specs=pl.BlockSpec((_BLK, H2), lambda n: (n, 0)),
        out_shape=jax.ShapeDtypeStruct((B, H2), jnp.float32),
    )(u, i, g, W1, b1.reshape(1, H1), W2, b2.reshape(1, H2))


def kernel(user_input, item_input, genre_input, user_table, item_table,
           genre_table, W1, b1, W2, b2):
    u, i, g = _SC_GATHER(user_input, item_input, genre_input,
                         user_table, item_table, genre_table)
    return _tc_mlp(u, i, g, W1, b1, W2, b2)
